# baseline (speedup 1.0000x reference)
import functools

import jax
import jax.numpy as jnp
from jax import lax
from jax.experimental import pallas as pl
from jax.experimental.pallas import tpu as pltpu
from jax.experimental.pallas import tpu_sc as plsc

E = 320000
D = 128
U = 128
G = 64
H = 512
OUT = 384

BLK = 512

_dot = functools.partial(jnp.dot, preferred_element_type=jnp.float32)


def _ln(x, g, b):
    m = jnp.mean(x, axis=-1, keepdims=True)
    v = jnp.mean((x - m) ** 2, axis=-1, keepdims=True)
    return (x - m) * lax.rsqrt(v + 1e-5) * g + b


def _tc_body(src_ref, dest_ref, ea_ref, u_ref, eb_ref,
             W1_ref, b1_ref, g1_ref, be1_ref,
             W2_ref, b2_ref, g2_ref, be2_ref,
             W3_ref, b3_ref,
             wW1_ref, wb1_ref, wg1_ref, wbe1_ref, wW2_ref, wb2_ref,
             eo_ref, wts_ref, segsum_ref):
    i = pl.program_id(0)
    eb = eb_ref[...]
    onehot = (eb == lax.broadcasted_iota(jnp.int32, (BLK, G), 1)
              ).astype(jnp.bfloat16)
    ue = _dot(onehot, u_ref[...])
    ue_b = ue.astype(jnp.bfloat16)

    src = src_ref[...].astype(jnp.bfloat16)
    dest = dest_ref[...].astype(jnp.bfloat16)
    ea = ea_ref[...].astype(jnp.bfloat16)

    xw = (_dot(ea, wW1_ref[0:D, :]) + _dot(ue_b, wW1_ref[D:D + U, :])
          + wb1_ref[...])
    hw = jax.nn.relu(_ln(xw, wg1_ref[...], wbe1_ref[...])).astype(jnp.bfloat16)
    wts = _dot(hw, wW2_ref[...]) + wb2_ref[...]
    wts_ref[...] = wts

    ex = jnp.exp(wts)
    partial = lax.dot_general(onehot.astype(jnp.float32), ex,
                              (((0,), (0,)), ((), ())),
                              preferred_element_type=jnp.float32)

    @pl.when(i == 0)
    def _():
        segsum_ref[...] = jnp.zeros_like(segsum_ref)

    segsum_ref[...] += partial

    x1 = (_dot(src, W1_ref[0:D, :]) + _dot(dest, W1_ref[D:2 * D, :])
          + _dot(ea, W1_ref[2 * D:3 * D, :])
          + _dot(ue_b, W1_ref[3 * D:3 * D + U, :])
          + b1_ref[...])
    h1 = jax.nn.relu(_ln(x1, g1_ref[...], be1_ref[...])).astype(jnp.bfloat16)
    h2 = jax.nn.relu(_ln(_dot(h1, W2_ref[...]) + b2_ref[...],
                         g2_ref[...], be2_ref[...])).astype(jnp.bfloat16)
    eo_ref[...] = _dot(h2, W3_ref[...]) + b3_ref[...]


def _tc_call(src, dest, ea, u, eb2,
             W1, b1, g1, be1, W2, b2, g2, be2, W3, b3,
             wW1, wb1, wg1, wbe1, wW2, wb2):
    n_blk = E // BLK
    row = lambda i: (i, 0)
    const = lambda i: (0, 0)
    full = lambda a: pl.BlockSpec(a.shape, const)
    return pl.pallas_call(
        _tc_body,
        grid=(n_blk,),
        in_specs=[
            pl.BlockSpec((BLK, D), row),
            pl.BlockSpec((BLK, D), row),
            pl.BlockSpec((BLK, D), row),
            full(u),
            pl.BlockSpec((BLK, 1), row),
            full(W1), full(b1), full(g1), full(be1),
            full(W2), full(b2), full(g2), full(be2),
            full(W3), full(b3),
            full(wW1), full(wb1), full(wg1), full(wbe1),
            full(wW2), full(wb2),
        ],
        out_specs=[
            pl.BlockSpec((BLK, OUT), row),
            pl.BlockSpec((BLK, 1), row),
            pl.BlockSpec((G, 1), const),
        ],
        out_shape=[
            jax.ShapeDtypeStruct((E, OUT), jnp.float32),
            jax.ShapeDtypeStruct((E, 1), jnp.float32),
            jax.ShapeDtypeStruct((G, 1), jnp.float32),
        ],
        compiler_params=pltpu.CompilerParams(
            dimension_semantics=("arbitrary",),
        ),
    )(src, dest, ea, u, eb2,
      W1, b1, g1, be1, W2, b2, g2, be2, W3, b3,
      wW1, wb1, wg1, wbe1, wW2, wb2)


_NC, _NS, _LANES = 2, 16, 16
_NW = _NC * _NS
_CHUNK = E // _NW


def _sc_body(wts_hbm, eb_hbm, segsum_hbm, out_hbm, w_v, id_v, inv_v, out_v):
    wid = lax.axis_index("s") * _NC + lax.axis_index("c")
    base = wid * _CHUNK
    pltpu.sync_copy(wts_hbm.at[pl.ds(base, _CHUNK)], w_v)
    pltpu.sync_copy(eb_hbm.at[pl.ds(base, _CHUNK)], id_v)
    pltpu.sync_copy(segsum_hbm, inv_v)
    for j in range(G // _LANES):
        sl = pl.ds(j * _LANES, _LANES)
        inv_v[sl] = 1.0 / inv_v[sl]

    def body(t, carry):
        sl = pl.ds(t * _LANES, _LANES)
        ids = id_v[sl]
        w = w_v[sl]
        s = plsc.load_gather(inv_v, [ids])
        out_v[sl] = jnp.exp(w) * s
        return carry

    lax.fori_loop(0, _CHUNK // _LANES, body, 0)
    pltpu.sync_copy(out_v, out_hbm.at[pl.ds(base, _CHUNK)])


@functools.cache
def _sc_softmax():
    return pl.kernel(
        _sc_body,
        out_type=jax.ShapeDtypeStruct((E,), jnp.float32),
        mesh=plsc.VectorSubcoreMesh(core_axis_name="c", subcore_axis_name="s"),
        scratch_types=[
            pltpu.VMEM((_CHUNK,), jnp.float32),
            pltpu.VMEM((_CHUNK,), jnp.int32),
            pltpu.VMEM((G,), jnp.float32),
            pltpu.VMEM((_CHUNK,), jnp.float32),
        ],
        compiler_params=pltpu.CompilerParams(needs_layout_passes=False),
    )


def kernel(src, dest, edge_attr, u, edge_batch,
           em_W1, em_b1, em_g1, em_be1, em_W2, em_b2, em_g2, em_be2,
           em_W3, em_b3,
           wm_W1, wm_b1, wm_g1, wm_be1, wm_W2, wm_b2):
    eb = edge_batch.astype(jnp.int32)
    eb2 = eb.reshape(E, 1)
    r = lambda a: a.reshape(1, -1)
    b = lambda a: a.astype(jnp.bfloat16)
    edge_out, wts, segsum = _tc_call(
        src, dest, edge_attr, b(u), eb2,
        b(em_W1), r(em_b1), r(em_g1), r(em_be1),
        b(em_W2), r(em_b2), r(em_g2), r(em_be2),
        b(em_W3), r(em_b3),
        b(wm_W1), r(wm_b1), r(wm_g1), r(wm_be1),
        b(wm_W2), r(wm_b2))
    norm = _sc_softmax()(wts.reshape(E), eb, segsum.reshape(G))
    return edge_out, wts, norm.reshape(E, 1)


# speedup vs baseline: 1.2509x; 1.2509x over previous
import functools

import jax
import jax.numpy as jnp
from jax import lax
from jax.experimental import pallas as pl
from jax.experimental.pallas import tpu as pltpu
from jax.experimental.pallas import tpu_sc as plsc

E = 320000
D = 128
U = 128
G = 64
H = 512
OUT = 384

BLK = 1280

_dot = functools.partial(jnp.dot, preferred_element_type=jnp.float32)


def _ln(x, g, b):
    m = jnp.mean(x, axis=-1, keepdims=True)
    v = jnp.mean((x - m) ** 2, axis=-1, keepdims=True)
    return (x - m) * lax.rsqrt(v + 1e-5) * g + b


def _tc_body(src_ref, dest_ref, ea_ref, u_ref, eb_ref,
             W1_ref, b1_ref, g1_ref, be1_ref,
             W2_ref, b2_ref, g2_ref, be2_ref,
             W3_ref, b3_ref,
             wW1_ref, wb1_ref, wg1_ref, wbe1_ref, wW2_ref, wb2_ref,
             eo_ref, wts_ref, segsum_ref):
    i = pl.program_id(0)
    eb = eb_ref[...]
    onehot = (eb == lax.broadcasted_iota(jnp.int32, (BLK, G), 1)
              ).astype(jnp.float32)
    ue = _dot(onehot, u_ref[...])

    src = src_ref[...]
    dest = dest_ref[...]
    ea = ea_ref[...]

    xw = (_dot(ea, wW1_ref[0:D, :]) + _dot(ue, wW1_ref[D:D + U, :])
          + wb1_ref[...])
    hw = jax.nn.relu(_ln(xw, wg1_ref[...], wbe1_ref[...]))
    wts = _dot(hw, wW2_ref[...]) + wb2_ref[...]
    wts_ref[...] = wts

    ex = jnp.exp(wts)
    partial = lax.dot_general(onehot, ex, (((0,), (0,)), ((), ())),
                              preferred_element_type=jnp.float32)

    @pl.when(i == 0)
    def _():
        segsum_ref[...] = jnp.zeros_like(segsum_ref)

    segsum_ref[...] += partial

    x1 = (_dot(src, W1_ref[0:D, :]) + _dot(dest, W1_ref[D:2 * D, :])
          + _dot(ea, W1_ref[2 * D:3 * D, :])
          + _dot(ue, W1_ref[3 * D:3 * D + U, :])
          + b1_ref[...])
    h1 = jax.nn.relu(_ln(x1, g1_ref[...], be1_ref[...]))
    h2 = jax.nn.relu(_ln(_dot(h1, W2_ref[...]) + b2_ref[...],
                         g2_ref[...], be2_ref[...]))
    eo_ref[...] = _dot(h2, W3_ref[...]) + b3_ref[...]


def _tc_call(src, dest, ea, u, eb2,
             W1, b1, g1, be1, W2, b2, g2, be2, W3, b3,
             wW1, wb1, wg1, wbe1, wW2, wb2):
    n_blk = E // BLK
    row = lambda i: (i, 0)
    const = lambda i: (0, 0)
    full = lambda a: pl.BlockSpec(a.shape, const)
    return pl.pallas_call(
        _tc_body,
        grid=(n_blk,),
        in_specs=[
            pl.BlockSpec((BLK, D), row),
            pl.BlockSpec((BLK, D), row),
            pl.BlockSpec((BLK, D), row),
            full(u),
            pl.BlockSpec((BLK, 1), row),
            full(W1), full(b1), full(g1), full(be1),
            full(W2), full(b2), full(g2), full(be2),
            full(W3), full(b3),
            full(wW1), full(wb1), full(wg1), full(wbe1),
            full(wW2), full(wb2),
        ],
        out_specs=[
            pl.BlockSpec((BLK, OUT), row),
            pl.BlockSpec((BLK, 1), row),
            pl.BlockSpec((G, 1), const),
        ],
        out_shape=[
            jax.ShapeDtypeStruct((E, OUT), jnp.float32),
            jax.ShapeDtypeStruct((E, 1), jnp.float32),
            jax.ShapeDtypeStruct((G, 1), jnp.float32),
        ],
        compiler_params=pltpu.CompilerParams(
            dimension_semantics=("arbitrary",),
        ),
    )(src, dest, ea, u, eb2,
      W1, b1, g1, be1, W2, b2, g2, be2, W3, b3,
      wW1, wb1, wg1, wbe1, wW2, wb2)


_NC, _NS, _LANES = 2, 16, 16
_NW = _NC * _NS
_CHUNK = E // _NW


def _sc_body(wts_hbm, eb_hbm, segsum_hbm, out_hbm, w_v, id_v, inv_v, out_v):
    wid = lax.axis_index("s") * _NC + lax.axis_index("c")
    base = wid * _CHUNK
    pltpu.sync_copy(wts_hbm.at[pl.ds(base, _CHUNK)], w_v)
    pltpu.sync_copy(eb_hbm.at[pl.ds(base, _CHUNK)], id_v)
    pltpu.sync_copy(segsum_hbm, inv_v)
    for j in range(G // _LANES):
        sl = pl.ds(j * _LANES, _LANES)
        inv_v[sl] = 1.0 / inv_v[sl]

    def body(t, carry):
        sl = pl.ds(t * _LANES, _LANES)
        ids = id_v[sl]
        w = w_v[sl]
        s = plsc.load_gather(inv_v, [ids])
        out_v[sl] = jnp.exp(w) * s
        return carry

    lax.fori_loop(0, _CHUNK // _LANES, body, 0)
    pltpu.sync_copy(out_v, out_hbm.at[pl.ds(base, _CHUNK)])


@functools.cache
def _sc_softmax():
    return pl.kernel(
        _sc_body,
        out_type=jax.ShapeDtypeStruct((E,), jnp.float32),
        mesh=plsc.VectorSubcoreMesh(core_axis_name="c", subcore_axis_name="s"),
        scratch_types=[
            pltpu.VMEM((_CHUNK,), jnp.float32),
            pltpu.VMEM((_CHUNK,), jnp.int32),
            pltpu.VMEM((G,), jnp.float32),
            pltpu.VMEM((_CHUNK,), jnp.float32),
        ],
        compiler_params=pltpu.CompilerParams(needs_layout_passes=False),
    )


def kernel(src, dest, edge_attr, u, edge_batch,
           em_W1, em_b1, em_g1, em_be1, em_W2, em_b2, em_g2, em_be2,
           em_W3, em_b3,
           wm_W1, wm_b1, wm_g1, wm_be1, wm_W2, wm_b2):
    eb = edge_batch.astype(jnp.int32)
    eb2 = eb.reshape(E, 1)
    r = lambda a: a.reshape(1, -1)
    edge_out, wts, segsum = _tc_call(
        src, dest, edge_attr, u, eb2,
        em_W1, r(em_b1), r(em_g1), r(em_be1),
        em_W2, r(em_b2), r(em_g2), r(em_be2),
        em_W3, r(em_b3),
        wm_W1, r(wm_b1), r(wm_g1), r(wm_be1),
        wm_W2, r(wm_b2))
    norm = _sc_softmax()(wts.reshape(E), eb, segsum.reshape(G))
    return edge_out, wts, norm.reshape(E, 1)


# speedup vs baseline: 1.3076x; 1.0453x over previous
import functools

import jax
import jax.numpy as jnp
from jax import lax
from jax.experimental import pallas as pl
from jax.experimental.pallas import tpu as pltpu
from jax.experimental.pallas import tpu_sc as plsc

E = 320000
D = 128
U = 128
G = 64
H = 512
OUT = 384

BLK = 2560

_dot = functools.partial(jnp.dot, preferred_element_type=jnp.float32)


def _ln(x, g, b):
    m = jnp.mean(x, axis=-1, keepdims=True)
    v = jnp.mean((x - m) ** 2, axis=-1, keepdims=True)
    return (x - m) * lax.rsqrt(v + 1e-5) * g + b


def _tc_body(src_ref, dest_ref, ea_ref, u_ref, eb_ref,
             W1_ref, b1_ref, g1_ref, be1_ref,
             W2_ref, b2_ref, g2_ref, be2_ref,
             W3_ref, b3_ref,
             wW1_ref, wb1_ref, wg1_ref, wbe1_ref, wW2_ref, wb2_ref,
             eo_ref, wts_ref, segsum_ref):
    i = pl.program_id(0)
    eb = eb_ref[...]
    onehot = (eb == lax.broadcasted_iota(jnp.int32, (BLK, G), 1)
              ).astype(jnp.float32)
    ue = _dot(onehot, u_ref[...])

    src = src_ref[...]
    dest = dest_ref[...]
    ea = ea_ref[...]

    xw = (_dot(ea, wW1_ref[0:D, :]) + _dot(ue, wW1_ref[D:D + U, :])
          + wb1_ref[...])
    hw = jax.nn.relu(_ln(xw, wg1_ref[...], wbe1_ref[...]))
    wts = _dot(hw, wW2_ref[...]) + wb2_ref[...]
    wts_ref[...] = wts

    ex = jnp.exp(wts)
    partial = lax.dot_general(onehot, ex, (((0,), (0,)), ((), ())),
                              preferred_element_type=jnp.float32)

    @pl.when(i == 0)
    def _():
        segsum_ref[...] = jnp.zeros_like(segsum_ref)

    segsum_ref[...] += partial

    x1 = (_dot(src, W1_ref[0:D, :]) + _dot(dest, W1_ref[D:2 * D, :])
          + _dot(ea, W1_ref[2 * D:3 * D, :])
          + _dot(ue, W1_ref[3 * D:3 * D + U, :])
          + b1_ref[...])
    h1 = jax.nn.relu(_ln(x1, g1_ref[...], be1_ref[...]))
    h2 = jax.nn.relu(_ln(_dot(h1, W2_ref[...]) + b2_ref[...],
                         g2_ref[...], be2_ref[...]))
    eo_ref[...] = _dot(h2, W3_ref[...]) + b3_ref[...]


def _tc_call(src, dest, ea, u, eb2,
             W1, b1, g1, be1, W2, b2, g2, be2, W3, b3,
             wW1, wb1, wg1, wbe1, wW2, wb2):
    n_blk = E // BLK
    row = lambda i: (i, 0)
    const = lambda i: (0, 0)
    full = lambda a: pl.BlockSpec(a.shape, const)
    return pl.pallas_call(
        _tc_body,
        grid=(n_blk,),
        in_specs=[
            pl.BlockSpec((BLK, D), row),
            pl.BlockSpec((BLK, D), row),
            pl.BlockSpec((BLK, D), row),
            full(u),
            pl.BlockSpec((BLK, 1), row),
            full(W1), full(b1), full(g1), full(be1),
            full(W2), full(b2), full(g2), full(be2),
            full(W3), full(b3),
            full(wW1), full(wb1), full(wg1), full(wbe1),
            full(wW2), full(wb2),
        ],
        out_specs=[
            pl.BlockSpec((BLK, OUT), row),
            pl.BlockSpec((BLK, 1), row),
            pl.BlockSpec((G, 1), const),
        ],
        out_shape=[
            jax.ShapeDtypeStruct((E, OUT), jnp.float32),
            jax.ShapeDtypeStruct((E, 1), jnp.float32),
            jax.ShapeDtypeStruct((G, 1), jnp.float32),
        ],
        compiler_params=pltpu.CompilerParams(
            dimension_semantics=("arbitrary",),
        ),
    )(src, dest, ea, u, eb2,
      W1, b1, g1, be1, W2, b2, g2, be2, W3, b3,
      wW1, wb1, wg1, wbe1, wW2, wb2)


_NC, _NS, _LANES = 2, 16, 16
_NW = _NC * _NS
_CHUNK = E // _NW


def _sc_body(wts_hbm, eb_hbm, segsum_hbm, out_hbm, w_v, id_v, inv_v, out_v):
    wid = lax.axis_index("s") * _NC + lax.axis_index("c")
    base = wid * _CHUNK
    pltpu.sync_copy(wts_hbm.at[pl.ds(base, _CHUNK)], w_v)
    pltpu.sync_copy(eb_hbm.at[pl.ds(base, _CHUNK)], id_v)
    pltpu.sync_copy(segsum_hbm, inv_v)
    for j in range(G // _LANES):
        sl = pl.ds(j * _LANES, _LANES)
        inv_v[sl] = 1.0 / inv_v[sl]

    def body(t, carry):
        sl = pl.ds(t * _LANES, _LANES)
        ids = id_v[sl]
        w = w_v[sl]
        s = plsc.load_gather(inv_v, [ids])
        out_v[sl] = jnp.exp(w) * s
        return carry

    lax.fori_loop(0, _CHUNK // _LANES, body, 0)
    pltpu.sync_copy(out_v, out_hbm.at[pl.ds(base, _CHUNK)])


@functools.cache
def _sc_softmax():
    return pl.kernel(
        _sc_body,
        out_type=jax.ShapeDtypeStruct((E,), jnp.float32),
        mesh=plsc.VectorSubcoreMesh(core_axis_name="c", subcore_axis_name="s"),
        scratch_types=[
            pltpu.VMEM((_CHUNK,), jnp.float32),
            pltpu.VMEM((_CHUNK,), jnp.int32),
            pltpu.VMEM((G,), jnp.float32),
            pltpu.VMEM((_CHUNK,), jnp.float32),
        ],
        compiler_params=pltpu.CompilerParams(needs_layout_passes=False),
    )


def kernel(src, dest, edge_attr, u, edge_batch,
           em_W1, em_b1, em_g1, em_be1, em_W2, em_b2, em_g2, em_be2,
           em_W3, em_b3,
           wm_W1, wm_b1, wm_g1, wm_be1, wm_W2, wm_b2):
    eb = edge_batch.astype(jnp.int32)
    eb2 = eb.reshape(E, 1)
    r = lambda a: a.reshape(1, -1)
    edge_out, wts, segsum = _tc_call(
        src, dest, edge_attr, u, eb2,
        em_W1, r(em_b1), r(em_g1), r(em_be1),
        em_W2, r(em_b2), r(em_g2), r(em_be2),
        em_W3, r(em_b3),
        wm_W1, r(wm_b1), r(wm_g1), r(wm_be1),
        wm_W2, r(wm_b2))
    norm = _sc_softmax()(wts.reshape(E), eb, segsum.reshape(G))
    return edge_out, wts, norm.reshape(E, 1)


# speedup vs baseline: 1.9630x; 1.5013x over previous
import functools

import jax
import jax.numpy as jnp
from jax import lax
from jax.experimental import pallas as pl
from jax.experimental.pallas import tpu as pltpu
from jax.experimental.pallas import tpu_sc as plsc

E = 320000
D = 128
U = 128
G = 64
H = 512
OUT = 384

BLK = 2560

_dot = functools.partial(jnp.dot, preferred_element_type=jnp.float32)


def _ln(x, g, b):
    m = jnp.mean(x, axis=-1, keepdims=True)
    v = jnp.mean((x - m) ** 2, axis=-1, keepdims=True)
    return (x - m) * lax.rsqrt(v + 1e-5) * g + b


def _tc_body(src_ref, dest_ref, ea_ref, u_ref, eb_ref,
             W1_ref, b1_ref, g1_ref, be1_ref,
             W2_ref, b2_ref, g2_ref, be2_ref,
             W3_ref, b3_ref,
             wW1_ref, wb1_ref, wg1_ref, wbe1_ref, wW2_ref, wb2_ref,
             eo_ref, wts_ref, segsum_ref):
    i = pl.program_id(0)
    eb = eb_ref[...]
    onehot = (eb == lax.broadcasted_iota(jnp.int32, (BLK, G), 1)
              ).astype(jnp.float32)
    ue = _dot(onehot, u_ref[...])

    src = src_ref[...]
    dest = dest_ref[...]
    ea = ea_ref[...]

    X = jnp.concatenate([src, dest, ea, ue], axis=1)

    xw = _dot(X, wW1_ref[...]) + wb1_ref[...]
    hw = jax.nn.relu(_ln(xw, wg1_ref[...], wbe1_ref[...]))
    wts = _dot(hw, wW2_ref[...]) + wb2_ref[...]
    wts_ref[...] = wts

    ex = jnp.exp(wts)
    partial = lax.dot_general(onehot, ex, (((0,), (0,)), ((), ())),
                              preferred_element_type=jnp.float32)

    @pl.when(i == 0)
    def _():
        segsum_ref[...] = jnp.zeros_like(segsum_ref)

    segsum_ref[...] += partial

    x1 = _dot(X, W1_ref[...]) + b1_ref[...]
    h1 = jax.nn.relu(_ln(x1, g1_ref[...], be1_ref[...]))
    h2 = jax.nn.relu(_ln(_dot(h1, W2_ref[...]) + b2_ref[...],
                         g2_ref[...], be2_ref[...]))
    eo_ref[...] = _dot(h2, W3_ref[...]) + b3_ref[...]


def _tc_call(src, dest, ea, u, eb2,
             W1, b1, g1, be1, W2, b2, g2, be2, W3, b3,
             wW1, wb1, wg1, wbe1, wW2, wb2):
    n_blk = E // BLK
    row = lambda i: (i, 0)
    const = lambda i: (0, 0)
    full = lambda a: pl.BlockSpec(a.shape, const)
    return pl.pallas_call(
        _tc_body,
        grid=(n_blk,),
        in_specs=[
            pl.BlockSpec((BLK, D), row),
            pl.BlockSpec((BLK, D), row),
            pl.BlockSpec((BLK, D), row),
            full(u),
            pl.BlockSpec((BLK, 1), row),
            full(W1), full(b1), full(g1), full(be1),
            full(W2), full(b2), full(g2), full(be2),
            full(W3), full(b3),
            full(wW1), full(wb1), full(wg1), full(wbe1),
            full(wW2), full(wb2),
        ],
        out_specs=[
            pl.BlockSpec((BLK, OUT), row),
            pl.BlockSpec((BLK, 1), row),
            pl.BlockSpec((G, 1), const),
        ],
        out_shape=[
            jax.ShapeDtypeStruct((E, OUT), jnp.float32),
            jax.ShapeDtypeStruct((E, 1), jnp.float32),
            jax.ShapeDtypeStruct((G, 1), jnp.float32),
        ],
        compiler_params=pltpu.CompilerParams(
            dimension_semantics=("arbitrary",),
        ),
    )(src, dest, ea, u, eb2,
      W1, b1, g1, be1, W2, b2, g2, be2, W3, b3,
      wW1, wb1, wg1, wbe1, wW2, wb2)


_NC, _NS, _LANES = 2, 16, 16
_NW = _NC * _NS
_CHUNK = E // _NW


def _sc_body(wts_hbm, eb_hbm, segsum_hbm, out_hbm, w_v, id_v, inv_v, out_v):
    wid = lax.axis_index("s") * _NC + lax.axis_index("c")
    base = wid * _CHUNK
    pltpu.sync_copy(wts_hbm.at[pl.ds(base, _CHUNK)], w_v)
    pltpu.sync_copy(eb_hbm.at[pl.ds(base, _CHUNK)], id_v)
    pltpu.sync_copy(segsum_hbm, inv_v)
    for j in range(G // _LANES):
        sl = pl.ds(j * _LANES, _LANES)
        inv_v[sl] = 1.0 / inv_v[sl]

    def body(t, carry):
        sl = pl.ds(t * _LANES, _LANES)
        ids = id_v[sl]
        w = w_v[sl]
        s = plsc.load_gather(inv_v, [ids])
        out_v[sl] = jnp.exp(w) * s
        return carry

    lax.fori_loop(0, _CHUNK // _LANES, body, 0)
    pltpu.sync_copy(out_v, out_hbm.at[pl.ds(base, _CHUNK)])


@functools.cache
def _sc_softmax():
    return pl.kernel(
        _sc_body,
        out_type=jax.ShapeDtypeStruct((E,), jnp.float32),
        mesh=plsc.VectorSubcoreMesh(core_axis_name="c", subcore_axis_name="s"),
        scratch_types=[
            pltpu.VMEM((_CHUNK,), jnp.float32),
            pltpu.VMEM((_CHUNK,), jnp.int32),
            pltpu.VMEM((G,), jnp.float32),
            pltpu.VMEM((_CHUNK,), jnp.float32),
        ],
        compiler_params=pltpu.CompilerParams(needs_layout_passes=False),
    )


def kernel(src, dest, edge_attr, u, edge_batch,
           em_W1, em_b1, em_g1, em_be1, em_W2, em_b2, em_g2, em_be2,
           em_W3, em_b3,
           wm_W1, wm_b1, wm_g1, wm_be1, wm_W2, wm_b2):
    eb = edge_batch.astype(jnp.int32)
    eb2 = eb.reshape(E, 1)
    r = lambda a: a.reshape(1, -1)
    wm_W1p = jnp.concatenate(
        [jnp.zeros((2 * D, H), jnp.float32), wm_W1], axis=0)
    edge_out, wts, segsum = _tc_call(
        src, dest, edge_attr, u, eb2,
        em_W1, r(em_b1), r(em_g1), r(em_be1),
        em_W2, r(em_b2), r(em_g2), r(em_be2),
        em_W3, r(em_b3),
        wm_W1p, r(wm_b1), r(wm_g1), r(wm_be1),
        wm_W2, r(wm_b2))
    norm = _sc_softmax()(wts.reshape(E), eb, segsum.reshape(G))
    return edge_out, wts, norm.reshape(E, 1)


# speedup vs baseline: 2.0465x; 1.0425x over previous
import functools

import jax
import jax.numpy as jnp
from jax import lax
from jax.experimental import pallas as pl
from jax.experimental.pallas import tpu as pltpu
from jax.experimental.pallas import tpu_sc as plsc

E = 320000
D = 128
U = 128
G = 64
H = 512
OUT = 384

BLK = 2560

_dot = functools.partial(jnp.dot, preferred_element_type=jnp.float32)


def _ln(x, g, b):
    m = jnp.mean(x, axis=-1, keepdims=True)
    v = jnp.mean((x - m) ** 2, axis=-1, keepdims=True)
    return (x - m) * lax.rsqrt(v + 1e-5) * g + b


def _tc_body(src_ref, dest_ref, ea_ref, u_ref, eb_ref,
             W1_ref, b1_ref, g1_ref, be1_ref,
             W2_ref, b2_ref, g2_ref, be2_ref,
             W3_ref, b3_ref,
             wW1_ref, wb1_ref, wg1_ref, wbe1_ref, wW2_ref, wb2_ref,
             eo_ref, wts_ref, segsum_ref):
    i = pl.program_id(0)
    eb = eb_ref[...]
    onehot = (eb == lax.broadcasted_iota(jnp.int32, (BLK, G), 1)
              ).astype(jnp.float32)
    ue = _dot(onehot, u_ref[...])

    src = src_ref[...]
    dest = dest_ref[...]
    ea = ea_ref[...]

    X = jnp.concatenate([src, dest, ea, ue], axis=1)

    xw = _dot(X[:, 2 * D:], wW1_ref[...]) + wb1_ref[...]
    hw = jax.nn.relu(_ln(xw, wg1_ref[...], wbe1_ref[...]))
    wts = _dot(hw, wW2_ref[...]) + wb2_ref[...]
    wts_ref[...] = wts

    ex = jnp.exp(wts)
    partial = lax.dot_general(onehot, ex, (((0,), (0,)), ((), ())),
                              preferred_element_type=jnp.float32)

    @pl.when(i == 0)
    def _():
        segsum_ref[...] = jnp.zeros_like(segsum_ref)

    segsum_ref[...] += partial

    x1 = _dot(X, W1_ref[...]) + b1_ref[...]
    h1 = jax.nn.relu(_ln(x1, g1_ref[...], be1_ref[...]))
    h2 = jax.nn.relu(_ln(_dot(h1, W2_ref[...]) + b2_ref[...],
                         g2_ref[...], be2_ref[...]))
    eo_ref[...] = _dot(h2, W3_ref[...]) + b3_ref[...]


def _tc_call(src, dest, ea, u, eb2,
             W1, b1, g1, be1, W2, b2, g2, be2, W3, b3,
             wW1, wb1, wg1, wbe1, wW2, wb2):
    n_blk = E // BLK
    row = lambda i: (i, 0)
    const = lambda i: (0, 0)
    full = lambda a: pl.BlockSpec(a.shape, const)
    return pl.pallas_call(
        _tc_body,
        grid=(n_blk,),
        in_specs=[
            pl.BlockSpec((BLK, D), row),
            pl.BlockSpec((BLK, D), row),
            pl.BlockSpec((BLK, D), row),
            full(u),
            pl.BlockSpec((BLK, 1), row),
            full(W1), full(b1), full(g1), full(be1),
            full(W2), full(b2), full(g2), full(be2),
            full(W3), full(b3),
            full(wW1), full(wb1), full(wg1), full(wbe1),
            full(wW2), full(wb2),
        ],
        out_specs=[
            pl.BlockSpec((BLK, OUT), row),
            pl.BlockSpec((BLK, 1), row),
            pl.BlockSpec((G, 1), const),
        ],
        out_shape=[
            jax.ShapeDtypeStruct((E, OUT), jnp.float32),
            jax.ShapeDtypeStruct((E, 1), jnp.float32),
            jax.ShapeDtypeStruct((G, 1), jnp.float32),
        ],
        compiler_params=pltpu.CompilerParams(
            dimension_semantics=("arbitrary",),
        ),
    )(src, dest, ea, u, eb2,
      W1, b1, g1, be1, W2, b2, g2, be2, W3, b3,
      wW1, wb1, wg1, wbe1, wW2, wb2)


_NC, _NS, _LANES = 2, 16, 16
_NW = _NC * _NS
_CHUNK = E // _NW


def _sc_body(wts_hbm, eb_hbm, segsum_hbm, out_hbm, w_v, id_v, inv_v, out_v):
    wid = lax.axis_index("s") * _NC + lax.axis_index("c")
    base = wid * _CHUNK
    pltpu.sync_copy(wts_hbm.at[pl.ds(base, _CHUNK)], w_v)
    pltpu.sync_copy(eb_hbm.at[pl.ds(base, _CHUNK)], id_v)
    pltpu.sync_copy(segsum_hbm, inv_v)
    for j in range(G // _LANES):
        sl = pl.ds(j * _LANES, _LANES)
        inv_v[sl] = 1.0 / inv_v[sl]

    def body(t, carry):
        sl = pl.ds(t * _LANES, _LANES)
        ids = id_v[sl]
        w = w_v[sl]
        s = plsc.load_gather(inv_v, [ids])
        out_v[sl] = jnp.exp(w) * s
        return carry

    lax.fori_loop(0, _CHUNK // _LANES, body, 0)
    pltpu.sync_copy(out_v, out_hbm.at[pl.ds(base, _CHUNK)])


@functools.cache
def _sc_softmax():
    return pl.kernel(
        _sc_body,
        out_type=jax.ShapeDtypeStruct((E,), jnp.float32),
        mesh=plsc.VectorSubcoreMesh(core_axis_name="c", subcore_axis_name="s"),
        scratch_types=[
            pltpu.VMEM((_CHUNK,), jnp.float32),
            pltpu.VMEM((_CHUNK,), jnp.int32),
            pltpu.VMEM((G,), jnp.float32),
            pltpu.VMEM((_CHUNK,), jnp.float32),
        ],
        compiler_params=pltpu.CompilerParams(needs_layout_passes=False),
    )


def kernel(src, dest, edge_attr, u, edge_batch,
           em_W1, em_b1, em_g1, em_be1, em_W2, em_b2, em_g2, em_be2,
           em_W3, em_b3,
           wm_W1, wm_b1, wm_g1, wm_be1, wm_W2, wm_b2):
    eb = edge_batch.astype(jnp.int32)
    eb2 = eb.reshape(E, 1)
    r = lambda a: a.reshape(1, -1)
    edge_out, wts, segsum = _tc_call(
        src, dest, edge_attr, u, eb2,
        em_W1, r(em_b1), r(em_g1), r(em_be1),
        em_W2, r(em_b2), r(em_g2), r(em_be2),
        em_W3, r(em_b3),
        wm_W1, r(wm_b1), r(wm_g1), r(wm_be1),
        wm_W2, r(wm_b2))
    norm = _sc_softmax()(wts.reshape(E), eb, segsum.reshape(G))
    return edge_out, wts, norm.reshape(E, 1)


# speedup vs baseline: 2.1197x; 1.0358x over previous
import functools

import jax
import jax.numpy as jnp
from jax import lax
from jax.experimental import pallas as pl
from jax.experimental.pallas import tpu as pltpu
from jax.experimental.pallas import tpu_sc as plsc

E = 320000
D = 128
U = 128
G = 64
H = 512
OUT = 384

BLK = 3200

_dot = functools.partial(jnp.dot, preferred_element_type=jnp.float32)


def _ln(x, g, b):
    m = jnp.mean(x, axis=-1, keepdims=True)
    v = jnp.mean(x * x, axis=-1, keepdims=True) - m * m
    return (x - m) * lax.rsqrt(v + 1e-5) * g + b


def _tc_body(src_ref, dest_ref, ea_ref, u_ref, eb_ref,
             W1_ref, b1_ref, g1_ref, be1_ref,
             W2_ref, b2_ref, g2_ref, be2_ref,
             W3_ref, b3_ref,
             wW1_ref, wb1_ref, wg1_ref, wbe1_ref, wW2_ref, wb2_ref,
             eo_ref, wts_ref, segsum_ref):
    i = pl.program_id(0)
    eb = eb_ref[...]
    onehot = (eb == lax.broadcasted_iota(jnp.int32, (BLK, G), 1)
              ).astype(jnp.float32)
    ue = _dot(onehot, u_ref[...])

    src = src_ref[...]
    dest = dest_ref[...]
    ea = ea_ref[...]

    X = jnp.concatenate([src, dest, ea, ue], axis=1)

    xw = _dot(X[:, 2 * D:], wW1_ref[...]) + wb1_ref[...]
    hw = jax.nn.relu(_ln(xw, wg1_ref[...], wbe1_ref[...]))
    wts = _dot(hw, wW2_ref[...]) + wb2_ref[...]
    wts_ref[...] = wts

    ex = jnp.exp(wts)
    partial = lax.dot_general(onehot, ex, (((0,), (0,)), ((), ())),
                              preferred_element_type=jnp.float32)

    @pl.when(i == 0)
    def _():
        segsum_ref[...] = jnp.zeros_like(segsum_ref)

    segsum_ref[...] += partial

    x1 = _dot(X, W1_ref[...]) + b1_ref[...]
    h1 = jax.nn.relu(_ln(x1, g1_ref[...], be1_ref[...]))
    h2 = jax.nn.relu(_ln(_dot(h1, W2_ref[...]) + b2_ref[...],
                         g2_ref[...], be2_ref[...]))
    eo_ref[...] = _dot(h2, W3_ref[...]) + b3_ref[...]


def _tc_call(src, dest, ea, u, eb2,
             W1, b1, g1, be1, W2, b2, g2, be2, W3, b3,
             wW1, wb1, wg1, wbe1, wW2, wb2):
    n_blk = E // BLK
    row = lambda i: (i, 0)
    const = lambda i: (0, 0)
    full = lambda a: pl.BlockSpec(a.shape, const)
    return pl.pallas_call(
        _tc_body,
        grid=(n_blk,),
        in_specs=[
            pl.BlockSpec((BLK, D), row),
            pl.BlockSpec((BLK, D), row),
            pl.BlockSpec((BLK, D), row),
            full(u),
            pl.BlockSpec((BLK, 1), row),
            full(W1), full(b1), full(g1), full(be1),
            full(W2), full(b2), full(g2), full(be2),
            full(W3), full(b3),
            full(wW1), full(wb1), full(wg1), full(wbe1),
            full(wW2), full(wb2),
        ],
        out_specs=[
            pl.BlockSpec((BLK, OUT), row),
            pl.BlockSpec((BLK, 1), row),
            pl.BlockSpec((G, 1), const),
        ],
        out_shape=[
            jax.ShapeDtypeStruct((E, OUT), jnp.float32),
            jax.ShapeDtypeStruct((E, 1), jnp.float32),
            jax.ShapeDtypeStruct((G, 1), jnp.float32),
        ],
        compiler_params=pltpu.CompilerParams(
            dimension_semantics=("arbitrary",),
        ),
    )(src, dest, ea, u, eb2,
      W1, b1, g1, be1, W2, b2, g2, be2, W3, b3,
      wW1, wb1, wg1, wbe1, wW2, wb2)


_NC, _NS, _LANES = 2, 16, 16
_NW = _NC * _NS
_CHUNK = E // _NW


def _sc_body(wts_hbm, eb_hbm, segsum_hbm, out_hbm, w_v, id_v, inv_v, out_v):
    wid = lax.axis_index("s") * _NC + lax.axis_index("c")
    base = wid * _CHUNK
    pltpu.sync_copy(wts_hbm.at[pl.ds(base, _CHUNK)], w_v)
    pltpu.sync_copy(eb_hbm.at[pl.ds(base, _CHUNK)], id_v)
    pltpu.sync_copy(segsum_hbm, inv_v)
    for j in range(G // _LANES):
        sl = pl.ds(j * _LANES, _LANES)
        inv_v[sl] = 1.0 / inv_v[sl]

    def body(t, carry):
        sl = pl.ds(t * _LANES, _LANES)
        ids = id_v[sl]
        w = w_v[sl]
        s = plsc.load_gather(inv_v, [ids])
        out_v[sl] = jnp.exp(w) * s
        return carry

    lax.fori_loop(0, _CHUNK // _LANES, body, 0)
    pltpu.sync_copy(out_v, out_hbm.at[pl.ds(base, _CHUNK)])


@functools.cache
def _sc_softmax():
    return pl.kernel(
        _sc_body,
        out_type=jax.ShapeDtypeStruct((E,), jnp.float32),
        mesh=plsc.VectorSubcoreMesh(core_axis_name="c", subcore_axis_name="s"),
        scratch_types=[
            pltpu.VMEM((_CHUNK,), jnp.float32),
            pltpu.VMEM((_CHUNK,), jnp.int32),
            pltpu.VMEM((G,), jnp.float32),
            pltpu.VMEM((_CHUNK,), jnp.float32),
        ],
        compiler_params=pltpu.CompilerParams(needs_layout_passes=False),
    )


def kernel(src, dest, edge_attr, u, edge_batch,
           em_W1, em_b1, em_g1, em_be1, em_W2, em_b2, em_g2, em_be2,
           em_W3, em_b3,
           wm_W1, wm_b1, wm_g1, wm_be1, wm_W2, wm_b2):
    eb = edge_batch.astype(jnp.int32)
    eb2 = eb.reshape(E, 1)
    r = lambda a: a.reshape(1, -1)
    edge_out, wts, segsum = _tc_call(
        src, dest, edge_attr, u, eb2,
        em_W1, r(em_b1), r(em_g1), r(em_be1),
        em_W2, r(em_b2), r(em_g2), r(em_be2),
        em_W3, r(em_b3),
        wm_W1, r(wm_b1), r(wm_g1), r(wm_be1),
        wm_W2, r(wm_b2))
    norm = _sc_softmax()(wts.reshape(E), eb, segsum.reshape(G))
    return edge_out, wts, norm.reshape(E, 1)


# speedup vs baseline: 2.1422x; 1.0106x over previous
import functools

import jax
import jax.numpy as jnp
from jax import lax
from jax.experimental import pallas as pl
from jax.experimental.pallas import tpu as pltpu
from jax.experimental.pallas import tpu_sc as plsc

E = 320000
D = 128
U = 128
G = 64
H = 512
OUT = 384

BLK = 4000

_dot = functools.partial(jnp.dot, preferred_element_type=jnp.float32)


def _ln(x, g, b):
    m = jnp.mean(x, axis=-1, keepdims=True)
    v = jnp.mean(x * x, axis=-1, keepdims=True) - m * m
    return (x - m) * lax.rsqrt(v + 1e-5) * g + b


def _tc_body(src_ref, dest_ref, ea_ref, u_ref, eb_ref,
             W1_ref, b1_ref, g1_ref, be1_ref,
             W2_ref, b2_ref, g2_ref, be2_ref,
             W3_ref, b3_ref,
             wW1_ref, wb1_ref, wg1_ref, wbe1_ref, wW2_ref, wb2_ref,
             eo_ref, wts_ref, segsum_ref):
    i = pl.program_id(0)
    eb = eb_ref[...]
    onehot = (eb == lax.broadcasted_iota(jnp.int32, (BLK, G), 1)
              ).astype(jnp.float32)
    ue = _dot(onehot, u_ref[...])

    src = src_ref[...]
    dest = dest_ref[...]
    ea = ea_ref[...]

    X = jnp.concatenate([src, dest, ea, ue], axis=1)

    xw = _dot(X[:, 2 * D:], wW1_ref[...]) + wb1_ref[...]
    hw = jax.nn.relu(_ln(xw, wg1_ref[...], wbe1_ref[...]))
    wts = _dot(hw, wW2_ref[...]) + wb2_ref[...]
    wts_ref[...] = wts

    ex = jnp.exp(wts)
    partial = lax.dot_general(onehot, ex, (((0,), (0,)), ((), ())),
                              preferred_element_type=jnp.float32)

    @pl.when(i == 0)
    def _():
        segsum_ref[...] = jnp.zeros_like(segsum_ref)

    segsum_ref[...] += partial

    x1 = _dot(X, W1_ref[...]) + b1_ref[...]
    h1 = jax.nn.relu(_ln(x1, g1_ref[...], be1_ref[...]))
    h2 = jax.nn.relu(_ln(_dot(h1, W2_ref[...]) + b2_ref[...],
                         g2_ref[...], be2_ref[...]))
    eo_ref[...] = _dot(h2, W3_ref[...]) + b3_ref[...]


def _tc_call(src, dest, ea, u, eb2,
             W1, b1, g1, be1, W2, b2, g2, be2, W3, b3,
             wW1, wb1, wg1, wbe1, wW2, wb2):
    n_blk = E // BLK
    row = lambda i: (i, 0)
    const = lambda i: (0, 0)
    full = lambda a: pl.BlockSpec(a.shape, const)
    return pl.pallas_call(
        _tc_body,
        grid=(n_blk,),
        in_specs=[
            pl.BlockSpec((BLK, D), row),
            pl.BlockSpec((BLK, D), row),
            pl.BlockSpec((BLK, D), row),
            full(u),
            pl.BlockSpec((BLK, 1), row),
            full(W1), full(b1), full(g1), full(be1),
            full(W2), full(b2), full(g2), full(be2),
            full(W3), full(b3),
            full(wW1), full(wb1), full(wg1), full(wbe1),
            full(wW2), full(wb2),
        ],
        out_specs=[
            pl.BlockSpec((BLK, OUT), row),
            pl.BlockSpec((BLK, 1), row),
            pl.BlockSpec((G, 1), const),
        ],
        out_shape=[
            jax.ShapeDtypeStruct((E, OUT), jnp.float32),
            jax.ShapeDtypeStruct((E, 1), jnp.float32),
            jax.ShapeDtypeStruct((G, 1), jnp.float32),
        ],
        compiler_params=pltpu.CompilerParams(
            dimension_semantics=("arbitrary",),
        ),
    )(src, dest, ea, u, eb2,
      W1, b1, g1, be1, W2, b2, g2, be2, W3, b3,
      wW1, wb1, wg1, wbe1, wW2, wb2)


_NC, _NS, _LANES = 2, 16, 16
_NW = _NC * _NS
_CHUNK = E // _NW


def _sc_body(wts_hbm, eb_hbm, segsum_hbm, out_hbm, w_v, id_v, inv_v, out_v):
    wid = lax.axis_index("s") * _NC + lax.axis_index("c")
    base = wid * _CHUNK
    pltpu.sync_copy(wts_hbm.at[pl.ds(base, _CHUNK)], w_v)
    pltpu.sync_copy(eb_hbm.at[pl.ds(base, _CHUNK)], id_v)
    pltpu.sync_copy(segsum_hbm, inv_v)
    for j in range(G // _LANES):
        sl = pl.ds(j * _LANES, _LANES)
        inv_v[sl] = 1.0 / inv_v[sl]

    def body(t, carry):
        sl = pl.ds(t * _LANES, _LANES)
        ids = id_v[sl]
        w = w_v[sl]
        s = plsc.load_gather(inv_v, [ids])
        out_v[sl] = jnp.exp(w) * s
        return carry

    lax.fori_loop(0, _CHUNK // _LANES, body, 0)
    pltpu.sync_copy(out_v, out_hbm.at[pl.ds(base, _CHUNK)])


@functools.cache
def _sc_softmax():
    return pl.kernel(
        _sc_body,
        out_type=jax.ShapeDtypeStruct((E,), jnp.float32),
        mesh=plsc.VectorSubcoreMesh(core_axis_name="c", subcore_axis_name="s"),
        scratch_types=[
            pltpu.VMEM((_CHUNK,), jnp.float32),
            pltpu.VMEM((_CHUNK,), jnp.int32),
            pltpu.VMEM((G,), jnp.float32),
            pltpu.VMEM((_CHUNK,), jnp.float32),
        ],
        compiler_params=pltpu.CompilerParams(needs_layout_passes=False),
    )


def kernel(src, dest, edge_attr, u, edge_batch,
           em_W1, em_b1, em_g1, em_be1, em_W2, em_b2, em_g2, em_be2,
           em_W3, em_b3,
           wm_W1, wm_b1, wm_g1, wm_be1, wm_W2, wm_b2):
    eb = edge_batch.astype(jnp.int32)
    eb2 = eb.reshape(E, 1)
    r = lambda a: a.reshape(1, -1)
    edge_out, wts, segsum = _tc_call(
        src, dest, edge_attr, u, eb2,
        em_W1, r(em_b1), r(em_g1), r(em_be1),
        em_W2, r(em_b2), r(em_g2), r(em_be2),
        em_W3, r(em_b3),
        wm_W1, r(wm_b1), r(wm_g1), r(wm_be1),
        wm_W2, r(wm_b2))
    norm = _sc_softmax()(wts.reshape(E), eb, segsum.reshape(G))
    return edge_out, wts, norm.reshape(E, 1)


# speedup vs baseline: 2.1426x; 1.0002x over previous
import functools

import jax
import jax.numpy as jnp
from jax import lax
from jax.experimental import pallas as pl
from jax.experimental.pallas import tpu as pltpu
from jax.experimental.pallas import tpu_sc as plsc

E = 320000
D = 128
U = 128
G = 64
H = 512
OUT = 384

BLK = 4000

_dot = functools.partial(jnp.dot, preferred_element_type=jnp.float32)


def _ln(x, g, b):
    m = jnp.mean(x, axis=-1, keepdims=True)
    v = jnp.mean(x * x, axis=-1, keepdims=True) - m * m
    return (x - m) * lax.rsqrt(v + 1e-5) * g + b


def _tc_body(src_ref, dest_ref, ea_ref, u_ref, eb_ref,
             W1_ref, b1_ref, g1_ref, be1_ref,
             W2_ref, b2_ref, g2_ref, be2_ref,
             W3_ref, b3_ref,
             wW1_ref, wb1_ref, wg1_ref, wbe1_ref, wW2_ref, wb2_ref,
             eo_ref, wts_ref, segsum_ref):
    i = pl.program_id(0)
    eb = eb_ref[...]
    onehot = (eb == lax.broadcasted_iota(jnp.int32, (BLK, G), 1)
              ).astype(jnp.float32)
    ue = _dot(onehot, u_ref[...])

    src = src_ref[...]
    dest = dest_ref[...]
    ea = ea_ref[...]

    X = jnp.concatenate([src, dest, ea, ue], axis=1)

    xw = _dot(X[:, 2 * D:], wW1_ref[...]) + wb1_ref[...]
    hw = jax.nn.relu(_ln(xw, wg1_ref[...], wbe1_ref[...]))
    wts = _dot(hw, wW2_ref[...]) + wb2_ref[...]
    wts_ref[...] = wts

    ex = jnp.exp(wts)
    partial = lax.dot_general(onehot, ex, (((0,), (0,)), ((), ())),
                              preferred_element_type=jnp.float32)

    @pl.when(i == 0)
    def _():
        segsum_ref[...] = jnp.zeros_like(segsum_ref)

    segsum_ref[...] += partial

    x1 = _dot(X, W1_ref[...]) + b1_ref[...]
    h1 = jax.nn.relu(_ln(x1, g1_ref[...], be1_ref[...]))
    h2 = jax.nn.relu(_ln(_dot(h1, W2_ref[...]) + b2_ref[...],
                         g2_ref[...], be2_ref[...]))
    eo_ref[...] = _dot(h2, W3_ref[...]) + b3_ref[...]


def _tc_call(src, dest, ea, u, eb2,
             W1, b1, g1, be1, W2, b2, g2, be2, W3, b3,
             wW1, wb1, wg1, wbe1, wW2, wb2):
    n_blk = E // BLK
    row = lambda i: (i, 0)
    const = lambda i: (0, 0)
    full = lambda a: pl.BlockSpec(a.shape, const)
    return pl.pallas_call(
        _tc_body,
        grid=(n_blk,),
        in_specs=[
            pl.BlockSpec((BLK, D), row),
            pl.BlockSpec((BLK, D), row),
            pl.BlockSpec((BLK, D), row),
            full(u),
            pl.BlockSpec((BLK, 1), row),
            full(W1), full(b1), full(g1), full(be1),
            full(W2), full(b2), full(g2), full(be2),
            full(W3), full(b3),
            full(wW1), full(wb1), full(wg1), full(wbe1),
            full(wW2), full(wb2),
        ],
        out_specs=[
            pl.BlockSpec((BLK, OUT), row),
            pl.BlockSpec((BLK, 1), row),
            pl.BlockSpec((G, 1), const),
        ],
        out_shape=[
            jax.ShapeDtypeStruct((E, OUT), jnp.float32),
            jax.ShapeDtypeStruct((E, 1), jnp.float32),
            jax.ShapeDtypeStruct((G, 1), jnp.float32),
        ],
        compiler_params=pltpu.CompilerParams(
            dimension_semantics=("arbitrary",),
            vmem_limit_bytes=100 * 1024 * 1024,
        ),
    )(src, dest, ea, u, eb2,
      W1, b1, g1, be1, W2, b2, g2, be2, W3, b3,
      wW1, wb1, wg1, wbe1, wW2, wb2)


_NC, _NS, _LANES = 2, 16, 16
_NW = _NC * _NS
_CHUNK = E // _NW


def _sc_body(wts_hbm, eb_hbm, segsum_hbm, out_hbm, w_v, id_v, inv_v, out_v):
    wid = lax.axis_index("s") * _NC + lax.axis_index("c")
    base = wid * _CHUNK
    pltpu.sync_copy(wts_hbm.at[pl.ds(base, _CHUNK)], w_v)
    pltpu.sync_copy(eb_hbm.at[pl.ds(base, _CHUNK)], id_v)
    pltpu.sync_copy(segsum_hbm, inv_v)
    for j in range(G // _LANES):
        sl = pl.ds(j * _LANES, _LANES)
        inv_v[sl] = 1.0 / inv_v[sl]

    def body(t, carry):
        sl = pl.ds(t * _LANES, _LANES)
        ids = id_v[sl]
        w = w_v[sl]
        s = plsc.load_gather(inv_v, [ids])
        out_v[sl] = jnp.exp(w) * s
        return carry

    lax.fori_loop(0, _CHUNK // _LANES, body, 0)
    pltpu.sync_copy(out_v, out_hbm.at[pl.ds(base, _CHUNK)])


@functools.cache
def _sc_softmax():
    return pl.kernel(
        _sc_body,
        out_type=jax.ShapeDtypeStruct((E,), jnp.float32),
        mesh=plsc.VectorSubcoreMesh(core_axis_name="c", subcore_axis_name="s"),
        scratch_types=[
            pltpu.VMEM((_CHUNK,), jnp.float32),
            pltpu.VMEM((_CHUNK,), jnp.int32),
            pltpu.VMEM((G,), jnp.float32),
            pltpu.VMEM((_CHUNK,), jnp.float32),
        ],
        compiler_params=pltpu.CompilerParams(needs_layout_passes=False),
    )


def kernel(src, dest, edge_attr, u, edge_batch,
           em_W1, em_b1, em_g1, em_be1, em_W2, em_b2, em_g2, em_be2,
           em_W3, em_b3,
           wm_W1, wm_b1, wm_g1, wm_be1, wm_W2, wm_b2):
    eb = edge_batch.astype(jnp.int32)
    eb2 = eb.reshape(E, 1)
    r = lambda a: a.reshape(1, -1)
    edge_out, wts, segsum = _tc_call(
        src, dest, edge_attr, u, eb2,
        em_W1, r(em_b1), r(em_g1), r(em_be1),
        em_W2, r(em_b2), r(em_g2), r(em_be2),
        em_W3, r(em_b3),
        wm_W1, r(wm_b1), r(wm_g1), r(wm_be1),
        wm_W2, r(wm_b2))
    norm = _sc_softmax()(wts.reshape(E), eb, segsum.reshape(G))
    return edge_out, wts, norm.reshape(E, 1)


# speedup vs baseline: 2.1436x; 1.0005x over previous
import functools

import jax
import jax.numpy as jnp
from jax import lax
from jax.experimental import pallas as pl
from jax.experimental.pallas import tpu as pltpu
from jax.experimental.pallas import tpu_sc as plsc

E = 320000
D = 128
U = 128
G = 64
H = 512
OUT = 384

BLK = 4000

_dot = functools.partial(jnp.dot, preferred_element_type=jnp.float32)


def _ln(x, g, b):
    m = jnp.mean(x, axis=-1, keepdims=True)
    v = jnp.mean(x * x, axis=-1, keepdims=True) - m * m
    return (x - m) * lax.rsqrt(v + 1e-5) * g + b


def _tc_body(src_ref, dest_ref, ea_ref, u_ref, eb_ref,
             W1_ref, b1_ref, g1_ref, be1_ref,
             W2_ref, b2_ref, g2_ref, be2_ref,
             W3_ref, b3_ref,
             wW1_ref, wb1_ref, wg1_ref, wbe1_ref, wW2_ref, wb2_ref,
             eo_ref, wts_ref, segsum_ref):
    i = pl.program_id(0)
    eb = eb_ref[...]
    onehot = (eb == lax.broadcasted_iota(jnp.int32, (BLK, G), 1)
              ).astype(jnp.float32)
    ue = _dot(onehot, u_ref[...])

    src = src_ref[...]
    dest = dest_ref[...]
    ea = ea_ref[...]

    X = jnp.concatenate([src, dest, ea, ue], axis=1)

    xw = _dot(X[:, 2 * D:], wW1_ref[...]) + wb1_ref[...]
    hw = jax.nn.relu(_ln(xw, wg1_ref[...], wbe1_ref[...]))
    wts = _dot(hw, wW2_ref[...]) + wb2_ref[...]
    wts_ref[...] = wts

    ex = jnp.exp(wts)
    partial = lax.dot_general(onehot, ex, (((0,), (0,)), ((), ())),
                              preferred_element_type=jnp.float32)

    @pl.when(i == 0)
    def _():
        segsum_ref[...] = jnp.zeros_like(segsum_ref)

    segsum_ref[...] += partial

    x1 = _dot(X, W1_ref[...]) + b1_ref[...]
    h1 = jax.nn.relu(_ln(x1, g1_ref[...], be1_ref[...]))
    h2 = jax.nn.relu(_ln(_dot(h1, W2_ref[...]) + b2_ref[...],
                         g2_ref[...], be2_ref[...]))
    eo_ref[...] = _dot(h2, W3_ref[...]) + b3_ref[...]


def _tc_call(src, dest, ea, u, eb2,
             W1, b1, g1, be1, W2, b2, g2, be2, W3, b3,
             wW1, wb1, wg1, wbe1, wW2, wb2):
    n_blk = E // BLK
    row = lambda i: (i, 0)
    const = lambda i: (0, 0)
    full = lambda a: pl.BlockSpec(a.shape, const)
    return pl.pallas_call(
        _tc_body,
        grid=(n_blk,),
        in_specs=[
            pl.BlockSpec((BLK, D), row),
            pl.BlockSpec((BLK, D), row),
            pl.BlockSpec((BLK, D), row),
            full(u),
            pl.BlockSpec((BLK, 1), row),
            full(W1), full(b1), full(g1), full(be1),
            full(W2), full(b2), full(g2), full(be2),
            full(W3), full(b3),
            full(wW1), full(wb1), full(wg1), full(wbe1),
            full(wW2), full(wb2),
        ],
        out_specs=[
            pl.BlockSpec((BLK, OUT), row),
            pl.BlockSpec((BLK, 1), row),
            pl.BlockSpec((G, 1), const),
        ],
        out_shape=[
            jax.ShapeDtypeStruct((E, OUT), jnp.float32),
            jax.ShapeDtypeStruct((E, 1), jnp.float32),
            jax.ShapeDtypeStruct((G, 1), jnp.float32),
        ],
        compiler_params=pltpu.CompilerParams(
            dimension_semantics=("arbitrary",),
        ),
    )(src, dest, ea, u, eb2,
      W1, b1, g1, be1, W2, b2, g2, be2, W3, b3,
      wW1, wb1, wg1, wbe1, wW2, wb2)


_NC, _NS, _LANES = 2, 16, 16
_NW = _NC * _NS
_CHUNK = E // _NW


def _sc_body(wts_hbm, eb_hbm, segsum_hbm, out_hbm, w_v, id_v, inv_v, out_v):
    wid = lax.axis_index("s") * _NC + lax.axis_index("c")
    base = wid * _CHUNK
    pltpu.sync_copy(wts_hbm.at[pl.ds(base, _CHUNK)], w_v)
    pltpu.sync_copy(eb_hbm.at[pl.ds(base, _CHUNK)], id_v)
    pltpu.sync_copy(segsum_hbm, inv_v)
    for j in range(G // _LANES):
        sl = pl.ds(j * _LANES, _LANES)
        inv_v[sl] = 1.0 / inv_v[sl]

    def body(t, carry):
        sl = pl.ds(t * _LANES, _LANES)
        ids = id_v[sl]
        w = w_v[sl]
        s = plsc.load_gather(inv_v, [ids])
        out_v[sl] = jnp.exp(w) * s
        return carry

    lax.fori_loop(0, _CHUNK // _LANES, body, 0)
    pltpu.sync_copy(out_v, out_hbm.at[pl.ds(base, _CHUNK)])


@functools.cache
def _sc_softmax():
    return pl.kernel(
        _sc_body,
        out_type=jax.ShapeDtypeStruct((E,), jnp.float32),
        mesh=plsc.VectorSubcoreMesh(core_axis_name="c", subcore_axis_name="s"),
        scratch_types=[
            pltpu.VMEM((_CHUNK,), jnp.float32),
            pltpu.VMEM((_CHUNK,), jnp.int32),
            pltpu.VMEM((G,), jnp.float32),
            pltpu.VMEM((_CHUNK,), jnp.float32),
        ],
        compiler_params=pltpu.CompilerParams(needs_layout_passes=False),
    )


def kernel(src, dest, edge_attr, u, edge_batch,
           em_W1, em_b1, em_g1, em_be1, em_W2, em_b2, em_g2, em_be2,
           em_W3, em_b3,
           wm_W1, wm_b1, wm_g1, wm_be1, wm_W2, wm_b2):
    eb = edge_batch.astype(jnp.int32)
    eb2 = eb.reshape(E, 1)
    r = lambda a: a.reshape(1, -1)
    edge_out, wts, segsum = _tc_call(
        src, dest, edge_attr, u, eb2,
        em_W1, r(em_b1), r(em_g1), r(em_be1),
        em_W2, r(em_b2), r(em_g2), r(em_be2),
        em_W3, r(em_b3),
        wm_W1, r(wm_b1), r(wm_g1), r(wm_be1),
        wm_W2, r(wm_b2))
    norm = _sc_softmax()(wts.reshape(E), eb, segsum.reshape(G))
    return edge_out, wts, norm.reshape(E, 1)
